# baseline (speedup 1.0000x reference)
import functools

import jax
import jax.numpy as jnp
from jax import lax
from jax.experimental import pallas as pl
from jax.experimental.pallas import tpu as pltpu
from jax.experimental.pallas import tpu_sc as plsc

K = 5
B = 128
N = 32768
LANES = 16
PREFIX = 1024
UNROLL = 8


def _sc_workers():
    try:
        info = plsc.get_sparse_core_info()
        return info.num_cores, info.num_subcores
    except Exception:
        return 2, 16


def _make_sc_kernel(nc, ns):
    nw = nc * ns
    rows_per_w = B // nw
    mesh = plsc.VectorSubcoreMesh(core_axis_name="c", subcore_axis_name="s")

    @functools.partial(
        pl.kernel,
        mesh=mesh,
        compiler_params=pltpu.CompilerParams(needs_layout_passes=False),
        out_type=jax.ShapeDtypeStruct((nw, LANES), jnp.float32),
        scratch_types=[
            pltpu.VMEM((2 * LANES,), jnp.int32),
            pltpu.VMEM((rows_per_w, PREFIX), jnp.float32),
            pltpu.VMEM((N,), jnp.float32),
            pltpu.VMEM((rows_per_w, LANES), jnp.float32),
            pltpu.VMEM((LANES,), jnp.float32),
            [pltpu.SemaphoreType.DMA] * 8,
        ],
    )
    def sc_kernel(outputs_hbm, targets_hbm, out_hbm, t_vmem, pbuf, rbuf,
                  vbuf, res_vmem, sems):
        cid = lax.axis_index("c")
        sid = lax.axis_index("s")
        wid = sid * nc + cid
        r0 = wid * rows_per_w

        tbase = jnp.minimum((r0 // 8) * 8, B - LANES)
        tdesc = pltpu.async_copy(targets_hbm.at[pl.ds(tbase, LANES)],
                                 t_vmem.at[pl.ds(0, LANES)], sems[5])
        pdesc = pltpu.async_copy(
            outputs_hbm.at[pl.ds(r0, rows_per_w), pl.ds(0, PREFIX)], pbuf,
            sems[0])
        tdesc.wait()

        ts = []
        vdescs = []
        for j in range(rows_per_w):
            t = t_vmem[pl.ds(r0 + j - tbase, LANES)][0]
            talign = (t // LANES) * LANES
            ts.append(t)
            vdescs.append(pltpu.async_copy(
                outputs_hbm.at[r0 + j, pl.ds(talign, LANES)], vbuf.at[j],
                sems[rows_per_w + j]))

        iota = lax.iota(jnp.int32, LANES)
        zero16 = jnp.zeros((LANES,), jnp.int32)
        one16 = jnp.ones((LANES,), jnp.int32)
        acc = jnp.float32(0.0)
        for j in range(rows_per_w):
            r = r0 + j
            t = ts[j]
            t_vec = jnp.full((LANES,), t, jnp.int32)
            vdescs[j].wait()
            v_vec = plsc.load_gather(
                vbuf, [jnp.full((LANES,), j, jnp.int32),
                       t_vec - (t // LANES) * LANES])
            if j == 0:
                pdesc.wait()

            def p1_body(i, c, j=j, v_vec=v_vec):
                cs = list(c)
                for u in range(UNROLL):
                    x = pbuf[j, pl.ds(i * (LANES * UNROLL) + u * LANES, LANES)]
                    cs[u] = cs[u] + jnp.where(x > v_vec, one16, zero16)
                return tuple(cs)

            cgs = lax.fori_loop(0, PREFIX // (LANES * UNROLL), p1_body,
                                (zero16,) * UNROLL)
            cg = cgs[0]
            for u in range(1, UNROLL):
                cg = cg + cgs[u]
            sgt = jnp.sum(cg)

            def full_scan(v_vec=v_vec, t_vec=t_vec, r=r):
                pltpu.sync_copy(outputs_hbm.at[r], rbuf)

                def e_body(i, c):
                    x = rbuf[pl.ds(i * LANES, LANES)]
                    gidx = iota + i * LANES
                    pred = (x > v_vec) | ((x == v_vec) & (gidx < t_vec))
                    return c + jnp.where(pred, one16, zero16)

                rank = jnp.sum(lax.fori_loop(0, N // LANES, e_body, zero16))
                return jnp.where(rank < K, jnp.float32(1.0), jnp.float32(0.0))

            hit = lax.cond(sgt >= K, lambda: jnp.float32(0.0), full_scan)
            acc = acc + hit

        res_vmem[...] = jnp.where(iota == 0, acc, jnp.float32(0.0))
        pltpu.sync_copy(res_vmem, out_hbm.at[wid])

    return sc_kernel


def kernel(outputs, targets):
    nc, ns = _sc_workers()
    sc_kernel = _make_sc_kernel(nc, ns)
    partial = sc_kernel(outputs, targets.astype(jnp.int32))
    return jnp.sum(partial) * (100.0 / B)


# speedup vs baseline: 1.0052x; 1.0052x over previous
import functools

import jax
import jax.numpy as jnp
from jax import lax
from jax.experimental import pallas as pl
from jax.experimental.pallas import tpu as pltpu
from jax.experimental.pallas import tpu_sc as plsc

K = 5
B = 128
N = 32768
LANES = 16
PREFIX = 1024
UNROLL = 8


def _sc_workers():
    try:
        info = plsc.get_sparse_core_info()
        return info.num_cores, info.num_subcores
    except Exception:
        return 2, 16


def _make_sc_kernel(nc, ns):
    nw = nc * ns
    rows_per_w = B // nw
    mesh = plsc.VectorSubcoreMesh(core_axis_name="c", subcore_axis_name="s")

    @functools.partial(
        pl.kernel,
        mesh=mesh,
        compiler_params=pltpu.CompilerParams(needs_layout_passes=False),
        out_type=jax.ShapeDtypeStruct((nw, LANES), jnp.float32),
        scratch_types=[
            pltpu.VMEM((2 * LANES,), jnp.int32),
            pltpu.VMEM((rows_per_w, PREFIX), jnp.float32),
            pltpu.VMEM((N,), jnp.float32),
            pltpu.VMEM((rows_per_w, LANES), jnp.float32),
            pltpu.VMEM((LANES,), jnp.float32),
            [pltpu.SemaphoreType.DMA] * 8,
        ],
    )
    def sc_kernel(outputs_hbm, targets_hbm, out_hbm, t_vmem, pbuf, rbuf,
                  vbuf, res_vmem, sems):
        cid = lax.axis_index("c")
        sid = lax.axis_index("s")
        wid = sid * nc + cid
        r0 = wid * rows_per_w

        tbase = jnp.minimum((r0 // 8) * 8, B - LANES)
        tdesc = pltpu.async_copy(targets_hbm.at[pl.ds(tbase, LANES)],
                                 t_vmem.at[pl.ds(0, LANES)], sems[1])
        pdesc = pltpu.async_copy(
            outputs_hbm.at[pl.ds(r0, rows_per_w), pl.ds(0, PREFIX)], pbuf,
            sems[0])
        tdesc.wait()

        ts = []
        vdescs = []
        for j in range(rows_per_w):
            t = t_vmem[pl.ds(r0 + j - tbase, LANES)][0]
            talign = (t // LANES) * LANES
            ts.append(t)
            vdescs.append(pltpu.async_copy(
                outputs_hbm.at[r0 + j, pl.ds(talign, LANES)], vbuf.at[j],
                sems[rows_per_w + j]))

        iota = lax.iota(jnp.int32, LANES)
        zero16 = jnp.zeros((LANES,), jnp.int32)
        one16 = jnp.ones((LANES,), jnp.int32)
        acc = jnp.float32(0.0)
        for j in range(rows_per_w):
            r = r0 + j
            t = ts[j]
            t_vec = jnp.full((LANES,), t, jnp.int32)
            vdescs[j].wait()
            v_vec = plsc.load_gather(
                vbuf, [jnp.full((LANES,), j, jnp.int32),
                       t_vec - (t // LANES) * LANES])
            if j == 0:
                pdesc.wait()

            def p1_body(i, c, j=j, v_vec=v_vec):
                cs = list(c)
                for u in range(UNROLL):
                    x = pbuf[j, pl.ds(i * (LANES * UNROLL) + u * LANES, LANES)]
                    cs[u] = cs[u] + jnp.where(x > v_vec, one16, zero16)
                return tuple(cs)

            cgs = lax.fori_loop(0, PREFIX // (LANES * UNROLL), p1_body,
                                (zero16,) * UNROLL)
            cg = cgs[0]
            for u in range(1, UNROLL):
                cg = cg + cgs[u]
            sgt = jnp.sum(cg)

            def full_scan(v_vec=v_vec, t_vec=t_vec, r=r):
                pltpu.sync_copy(outputs_hbm.at[r], rbuf)

                def e_body(i, c):
                    x = rbuf[pl.ds(i * LANES, LANES)]
                    gidx = iota + i * LANES
                    pred = (x > v_vec) | ((x == v_vec) & (gidx < t_vec))
                    return c + jnp.where(pred, one16, zero16)

                rank = jnp.sum(lax.fori_loop(0, N // LANES, e_body, zero16))
                return jnp.where(rank < K, jnp.float32(1.0), jnp.float32(0.0))

            hit = lax.cond(sgt >= K, lambda: jnp.float32(0.0), full_scan)
            acc = acc + hit

        res_vmem[...] = jnp.where(iota == 0, acc, jnp.float32(0.0))
        pltpu.sync_copy(res_vmem, out_hbm.at[wid])

    return sc_kernel


def kernel(outputs, targets):
    nc, ns = _sc_workers()
    sc_kernel = _make_sc_kernel(nc, ns)
    partial = sc_kernel(outputs, targets.astype(jnp.int32))
    return jnp.sum(partial) * (100.0 / B)
